# Initial kernel scaffold; baseline (speedup 1.0000x reference)
import jax
import jax.numpy as jnp
from jax.experimental import pallas as pl


def kernel(x, Wq, bq, Wkv_down, bkv_down, Wk_up, bk_up, Wv_up, bv_up, Wq_idx, bq_idx, Wout, bout):
    raise NotImplementedError("write your pallas kernel here")


# speedup vs baseline: 204.9095x; 204.9095x over previous
import jax
import jax.numpy as jnp
from jax.experimental import pallas as pl

NUM_HEADS = 16
D_HEAD = 64
D_MODEL = NUM_HEADS * D_HEAD
D_LATENT = 128
TOP_K = 512
SEQ = 2048
BM = 256
NEG = -1e9


def _dot(a, b, trans_b=False):
    dn = (((1,), (1 if trans_b else 0,)), ((), ()))
    return jax.lax.dot_general(a, b, dn, preferred_element_type=jnp.float32)


def _proj_kernel(x_ref, wq_ref, bq_ref, wkv_ref, bkv_ref, wkup_ref, bkup_ref,
                 wvup_ref, bvup_ref, wqidx_ref, bqidx_ref,
                 q_ref, qidx_ref, kdown_ref, kup_ref, vup_ref):
    x = x_ref[...]
    q_ref[...] = _dot(x, wq_ref[...]) + bq_ref[...]
    qidx_ref[...] = _dot(x, wqidx_ref[...]) + bqidx_ref[...]
    kv = _dot(x, wkv_ref[...]) + bkv_ref[...]
    kd = kv[:, :D_LATENT]
    vd = kv[:, D_LATENT:]
    kdown_ref[...] = kd
    kup_ref[...] = _dot(kd, wkup_ref[...]) + bkup_ref[...]
    vup_ref[...] = _dot(vd, wvup_ref[...]) + bvup_ref[...]


def _attn_kernel(qidx_ref, kdown_ref, q_ref, kup_ref, vup_ref, wout_ref,
                 bout_ref, out_ref):
    qb = pl.program_id(0)
    qi = qidx_ref[...]
    kd = kdown_ref[...]
    fuzzy = jnp.maximum(_dot(qi, kd, trans_b=True), 0.0)

    rows = qb * BM + jax.lax.broadcasted_iota(jnp.int32, (BM, SEQ), 0)
    cols = jax.lax.broadcasted_iota(jnp.int32, (BM, SEQ), 1)
    causal = cols <= rows

    vi = jax.lax.bitcast_convert_type(fuzzy, jnp.int32)
    vi = jnp.where(causal, vi, -1)

    lo = jnp.full((BM, 1), -2, jnp.int32)
    hi = jnp.full((BM, 1), 0x7F800000, jnp.int32)
    for _ in range(32):
        mid = lo + (hi - lo) // 2
        cnt = jnp.sum(jnp.where(vi > mid, 1.0, 0.0), axis=1, keepdims=True)
        small = cnt < TOP_K
        hi = jnp.where(small, mid, hi)
        lo = jnp.where(small, lo, mid)
    cut = hi

    gt = vi > cut
    cnt_gt = jnp.sum(jnp.where(gt, 1.0, 0.0), axis=1, keepdims=True)
    rem = TOP_K - cnt_gt
    eq = (vi == cut) & (vi >= 0)

    lo2 = jnp.full((BM, 1), -1, jnp.int32)
    hi2 = jnp.full((BM, 1), SEQ - 1, jnp.int32)
    for _ in range(12):
        mid = lo2 + (hi2 - lo2) // 2
        cnt = jnp.sum(jnp.where(eq & (cols <= mid), 1.0, 0.0),
                      axis=1, keepdims=True)
        enough = cnt >= rem
        hi2 = jnp.where(enough, mid, hi2)
        lo2 = jnp.where(enough, lo2, mid)

    selected = gt | (eq & (cols <= hi2) & (rem > 0))
    bias = jnp.where(selected, 0.0, NEG)

    q = q_ref[...]
    ku = kup_ref[...]
    vu = vup_ref[...]
    scale = 1.0 / (D_HEAD ** 0.5)
    ctx = []
    for h in range(NUM_HEADS):
        sl = slice(h * D_HEAD, (h + 1) * D_HEAD)
        s = _dot(q[:, sl] * scale, ku[:, sl], trans_b=True) + bias
        m = jnp.max(s, axis=1, keepdims=True)
        e = jnp.exp(s - m)
        a = e / jnp.sum(e, axis=1, keepdims=True)
        ctx.append(_dot(a, vu[:, sl]))
    ctx = jnp.concatenate(ctx, axis=1)
    out_ref[...] = _dot(ctx, wout_ref[...]) + bout_ref[...]


def kernel(x, Wq, bq, Wkv_down, bkv_down, Wk_up, bk_up, Wv_up, bv_up,
           Wq_idx, bq_idx, Wout, bout):
    b, s, dm = x.shape
    x2 = x.reshape(s, dm)
    grid = (s // BM,)
    row_blk = lambda i: (i, 0)
    whole = lambda i: (0, 0)

    def full_spec(arr):
        return pl.BlockSpec(arr.shape, whole)

    b2 = lambda v: v.reshape(1, -1)

    q, qidx, kdown, kup, vup = pl.pallas_call(
        _proj_kernel,
        grid=grid,
        in_specs=[
            pl.BlockSpec((BM, dm), row_blk),
            full_spec(Wq), pl.BlockSpec((1, dm), whole),
            full_spec(Wkv_down), pl.BlockSpec((1, 2 * D_LATENT), whole),
            full_spec(Wk_up), pl.BlockSpec((1, dm), whole),
            full_spec(Wv_up), pl.BlockSpec((1, dm), whole),
            full_spec(Wq_idx), pl.BlockSpec((1, D_LATENT), whole),
        ],
        out_specs=[
            pl.BlockSpec((BM, dm), row_blk),
            pl.BlockSpec((BM, D_LATENT), row_blk),
            pl.BlockSpec((BM, D_LATENT), row_blk),
            pl.BlockSpec((BM, dm), row_blk),
            pl.BlockSpec((BM, dm), row_blk),
        ],
        out_shape=[
            jax.ShapeDtypeStruct((s, dm), jnp.float32),
            jax.ShapeDtypeStruct((s, D_LATENT), jnp.float32),
            jax.ShapeDtypeStruct((s, D_LATENT), jnp.float32),
            jax.ShapeDtypeStruct((s, dm), jnp.float32),
            jax.ShapeDtypeStruct((s, dm), jnp.float32),
        ],
    )(x2, Wq, b2(bq), Wkv_down, b2(bkv_down), Wk_up, b2(bk_up),
      Wv_up, b2(bv_up), Wq_idx, b2(bq_idx))

    out = pl.pallas_call(
        _attn_kernel,
        grid=grid,
        in_specs=[
            pl.BlockSpec((BM, D_LATENT), row_blk),
            pl.BlockSpec((s, D_LATENT), whole),
            pl.BlockSpec((BM, dm), row_blk),
            pl.BlockSpec((s, dm), whole),
            pl.BlockSpec((s, dm), whole),
            full_spec(Wout), pl.BlockSpec((1, dm), whole),
        ],
        out_specs=pl.BlockSpec((BM, dm), row_blk),
        out_shape=jax.ShapeDtypeStruct((s, dm), jnp.float32),
    )(qidx, kdown, q, kup, vup, Wout, b2(bout))

    return out.reshape(b, s, dm)
